# baseline (speedup 1.0000x reference)
import functools

import jax
import jax.numpy as jnp
from jax import lax
from jax.experimental import pallas as pl
from jax.experimental.pallas import tpu as pltpu
from jax.experimental.pallas import tpu_sc as plsc

N = 1_000_000
B = 4096
T = 4000
NT = N // T
G = T // 16
NW = 32
MAXM = -(-NT // NW)
NBUF = 3
MB = 65536


def _mse_tc(pred_t, tgt_t, gen_flag):
    grid = -(-N // MB)

    def body(p_ref, t_ref, g_ref, u_ref):
        d = p_ref[...] - t_ref[...]
        sq = d * d
        mse = sq[0, :] + sq[1, :] + sq[2, :]
        u_ref[...] = jnp.where(g_ref[...], mse, jnp.float32(-1.0))

    return pl.pallas_call(
        body,
        grid=(grid,),
        in_specs=[
            pl.BlockSpec((3, MB), lambda i: (0, i)),
            pl.BlockSpec((3, MB), lambda i: (0, i)),
            pl.BlockSpec((MB,), lambda i: (i,)),
        ],
        out_specs=pl.BlockSpec((MB,), lambda i: (i,)),
        out_shape=jax.ShapeDtypeStruct((N,), jnp.float32),
    )(pred_t, tgt_t, gen_flag)


def _sc_segment_partials(u, batch_idx):
    mesh = plsc.VectorSubcoreMesh(core_axis_name="c", subcore_axis_name="s")

    @functools.partial(
        pl.kernel,
        out_type=jax.ShapeDtypeStruct((128, 128), jnp.float32),
        mesh=mesh,
        compiler_params=pltpu.CompilerParams(needs_layout_passes=False),
        scratch_types=[
            pltpu.VMEM((T,), jnp.float32),
            pltpu.VMEM((T,), jnp.float32),
            pltpu.VMEM((T,), jnp.float32),
            pltpu.VMEM((T,), jnp.int32),
            pltpu.VMEM((T,), jnp.int32),
            pltpu.VMEM((T,), jnp.int32),
            pltpu.VMEM((17 * B,), jnp.float32),
            pltpu.VMEM((2, B), jnp.float32),
            pltpu.VMEM((16, 2, 256), jnp.float32),
            pltpu.VMEM((4, 128), jnp.float32),
            pltpu.VMEM_SHARED((16, 2, B), jnp.float32),
            pltpu.SemaphoreType.DMA,
            pltpu.SemaphoreType.DMA,
            pltpu.SemaphoreType.DMA,
        ],
    )
    def body(u_hbm, idx_hbm, out_hbm,
             vbuf0, vbuf1, vbuf2, ibuf0, ibuf1, ibuf2,
             acc, part, red_all, obuf, shared, sem0, sem1, sem2):
        cid = lax.axis_index("c")
        sid = lax.axis_index("s")
        wid = sid * 2 + cid

        bufs = ((vbuf0, ibuf0, sem0),
                (vbuf1, ibuf1, sem1),
                (vbuf2, ibuf2, sem2))

        it = lax.iota(jnp.int32, 16)
        it17 = it * 17
        mask8 = it < 8
        zeros16 = jnp.zeros((16,), jnp.float32)
        ones16 = jnp.ones((16,), jnp.float32)

        m_tiles = jnp.where(wid < NT - NW * (NT // NW), NT // NW + 1, NT // NW)

        def _copies(j, slot):
            vbuf, ibuf, sem = slot
            t = wid + NW * j
            r0 = pl.multiple_of(t * T, 8)
            return (
                pltpu.make_async_copy(u_hbm.at[pl.ds(r0, T)], vbuf, sem),
                pltpu.make_async_copy(idx_hbm.at[pl.ds(r0, T)], ibuf, sem),
            )

        def issue(j, slot):
            for c in _copies(j, slot):
                c.start()

        def drain(j, slot):
            for c in _copies(j, slot):
                c.wait()

        @plsc.parallel_loop(0, (17 * B) // 256, 1, unroll=2)
        def zacc(i):
            for r in range(16):
                acc[pl.ds((i * 16 + r) * 16, 16)] = zeros16

        def process(slot):
            vbuf, ibuf, _ = slot

            @plsc.parallel_loop(0, G, 1, unroll=4)
            def grp(i):
                d = pl.ds(i * 16, 16)
                idxv = ibuf[d]
                u = vbuf[d]
                wv = jnp.where(u >= 0.0, ones16, zeros16)
                v = jnp.maximum(u, 0.0)
                vr = jnp.flip(v, 0)
                wr = jnp.flip(wv, 0)
                ir = jnp.flip(idxv, 0)
                p1 = jnp.where(mask8, v, wr)
                c1 = jnp.where(mask8, idxv, ir)
                p2 = jnp.where(mask8, vr, wv)
                c2 = jnp.where(mask8, ir, idxv)
                plsc.addupdate_scatter(acc, [c1 * 17 + it], p1)
                plsc.addupdate_scatter(acc, [c2 * 17 + it], p2)

        @pl.when(0 < m_tiles)
        def _prime0():
            issue(0, bufs[0])

        @pl.when(1 < m_tiles)
        def _prime1():
            issue(1, bufs[1])

        def outer(k, carry):
            for b in range(NBUF):
                j = NBUF * k + b

                @pl.when(j < m_tiles)
                def _step():
                    @pl.when(j + 2 < m_tiles)
                    def _prefetch():
                        issue(j + 2, bufs[(b + 2) % NBUF])
                    drain(j, bufs[b])
                    process(bufs[b])
            return carry

        lax.fori_loop(0, -(-MAXM // NBUF), outer, 0)

        @plsc.parallel_loop(0, B // 16, 1, unroll=2)
        def lred(i):
            d = pl.ds(i * 16, 16)
            bv = i * 272 + it17
            s = plsc.load_gather(acc, [bv])
            for r in range(1, 8):
                s = s + plsc.load_gather(acc, [bv + r])
            c = plsc.load_gather(acc, [bv + 8])
            for r in range(9, 16):
                c = c + plsc.load_gather(acc, [bv + r])
            part[0, d] = s
            part[1, d] = c

        pltpu.sync_copy(part, shared.at[sid])
        plsc.subcore_barrier()

        off = pl.multiple_of(sid * 256, 8)
        for tt in range(16):
            pltpu.sync_copy(shared.at[tt, :, pl.ds(off, 256)],
                            red_all.at[tt])

        for rr in range(4):
            kind = rr // 2
            loff = (rr % 2) * 128

            @plsc.parallel_loop(0, 8, 1, unroll=2)
            def red(i):
                d2 = pl.ds(loff + i * 16, 16)
                s = red_all[0, kind, d2]
                for tt in range(1, 16):
                    s = s + red_all[tt, kind, d2]
                obuf[rr, pl.ds(i * 16, 16)] = s

        for kind in range(2):
            row0 = pl.multiple_of(cid * 64 + kind * 32 + sid * 2, 2)
            pltpu.sync_copy(obuf.at[pl.ds(kind * 2, 2), :],
                            out_hbm.at[pl.ds(row0, 2), :])

    return body(u, batch_idx)


def _finish(partials4):
    def fin(x_ref, o_ref):
        x = x_ref[...]
        s = x[0:32] + x[64:96]
        c = x[32:64] + x[96:128]
        loss = s / jnp.maximum(c, 1.0)
        o_ref[...] = (jnp.sum(loss) * (1.0 / B)).reshape(1, 1)

    return pl.pallas_call(
        fin,
        out_shape=jax.ShapeDtypeStruct((1, 1), jnp.float32),
    )(partials4)


def kernel(pred, tgt, t, gen_flag, batch_idx, gamma):
    del t, gamma
    u = _mse_tc(pred.T, tgt.T, gen_flag)
    partials = _sc_segment_partials(u, batch_idx)
    return _finish(partials)[0, 0]


# speedup vs baseline: 1.0674x; 1.0674x over previous
import functools

import jax
import jax.numpy as jnp
from jax import lax
from jax.experimental import pallas as pl
from jax.experimental.pallas import tpu as pltpu
from jax.experimental.pallas import tpu_sc as plsc

N = 1_000_000
B = 4096
T = 4000
NT = N // T
G = T // 16
NW = 32
MAXM = -(-NT // NW)
NBUF = 3
MB = 131072


def _mse_tc(pred_t, tgt_t, gen_flag):
    grid = -(-N // MB)

    def body(p_ref, t_ref, g_ref, u_ref):
        d = p_ref[...] - t_ref[...]
        sq = d * d
        mse = sq[0, :] + sq[1, :] + sq[2, :]
        u_ref[...] = jnp.where(g_ref[...] != 0, mse, jnp.float32(-1.0))

    return pl.pallas_call(
        body,
        grid=(grid,),
        in_specs=[
            pl.BlockSpec((3, MB), lambda i: (0, i)),
            pl.BlockSpec((3, MB), lambda i: (0, i)),
            pl.BlockSpec((MB,), lambda i: (i,)),
        ],
        out_specs=pl.BlockSpec((MB,), lambda i: (i,)),
        out_shape=jax.ShapeDtypeStruct((N,), jnp.float32),
    )(pred_t, tgt_t, gen_flag)


def _sc_segment_partials(u, batch_idx):
    mesh = plsc.VectorSubcoreMesh(core_axis_name="c", subcore_axis_name="s")

    @functools.partial(
        pl.kernel,
        out_type=jax.ShapeDtypeStruct((128, 128), jnp.float32),
        mesh=mesh,
        compiler_params=pltpu.CompilerParams(needs_layout_passes=False),
        scratch_types=[
            pltpu.VMEM((T,), jnp.float32),
            pltpu.VMEM((T,), jnp.float32),
            pltpu.VMEM((T,), jnp.float32),
            pltpu.VMEM((T,), jnp.int32),
            pltpu.VMEM((T,), jnp.int32),
            pltpu.VMEM((T,), jnp.int32),
            pltpu.VMEM((17 * B,), jnp.float32),
            pltpu.VMEM((2, B), jnp.float32),
            pltpu.VMEM((16, 2, 256), jnp.float32),
            pltpu.VMEM((4, 128), jnp.float32),
            pltpu.VMEM_SHARED((16, 2, B), jnp.float32),
            pltpu.SemaphoreType.DMA,
            pltpu.SemaphoreType.DMA,
            pltpu.SemaphoreType.DMA,
        ],
    )
    def body(u_hbm, idx_hbm, out_hbm,
             vbuf0, vbuf1, vbuf2, ibuf0, ibuf1, ibuf2,
             acc, part, red_all, obuf, shared, sem0, sem1, sem2):
        cid = lax.axis_index("c")
        sid = lax.axis_index("s")
        wid = sid * 2 + cid

        bufs = ((vbuf0, ibuf0, sem0),
                (vbuf1, ibuf1, sem1),
                (vbuf2, ibuf2, sem2))

        it = lax.iota(jnp.int32, 16)
        it17 = it * 17
        mask8 = it < 8
        zeros16 = jnp.zeros((16,), jnp.float32)
        ones16 = jnp.ones((16,), jnp.float32)

        m_tiles = jnp.where(wid < NT - NW * (NT // NW), NT // NW + 1, NT // NW)

        def _copies(j, slot):
            vbuf, ibuf, sem = slot
            t = wid + NW * j
            r0 = pl.multiple_of(t * T, 8)
            return (
                pltpu.make_async_copy(u_hbm.at[pl.ds(r0, T)], vbuf, sem),
                pltpu.make_async_copy(idx_hbm.at[pl.ds(r0, T)], ibuf, sem),
            )

        def issue(j, slot):
            for c in _copies(j, slot):
                c.start()

        def drain(j, slot):
            for c in _copies(j, slot):
                c.wait()

        @plsc.parallel_loop(0, (17 * B) // 256, 1, unroll=2)
        def zacc(i):
            for r in range(16):
                acc[pl.ds((i * 16 + r) * 16, 16)] = zeros16

        def process(slot):
            vbuf, ibuf, _ = slot

            @plsc.parallel_loop(0, G, 1, unroll=4)
            def grp(i):
                d = pl.ds(i * 16, 16)
                idxv = ibuf[d]
                u = vbuf[d]
                wv = jnp.where(u >= 0.0, ones16, zeros16)
                v = jnp.maximum(u, 0.0)
                vr = jnp.flip(v, 0)
                wr = jnp.flip(wv, 0)
                ir = jnp.flip(idxv, 0)
                p1 = jnp.where(mask8, v, wr)
                c1 = jnp.where(mask8, idxv, ir)
                p2 = jnp.where(mask8, vr, wv)
                c2 = jnp.where(mask8, ir, idxv)
                plsc.addupdate_scatter(acc, [c1 * 17 + it], p1)
                plsc.addupdate_scatter(acc, [c2 * 17 + it], p2)

        @pl.when(0 < m_tiles)
        def _prime0():
            issue(0, bufs[0])

        @pl.when(1 < m_tiles)
        def _prime1():
            issue(1, bufs[1])

        def outer(k, carry):
            for b in range(NBUF):
                j = NBUF * k + b

                @pl.when(j < m_tiles)
                def _step():
                    @pl.when(j + 2 < m_tiles)
                    def _prefetch():
                        issue(j + 2, bufs[(b + 2) % NBUF])
                    drain(j, bufs[b])
                    process(bufs[b])
            return carry

        lax.fori_loop(0, -(-MAXM // NBUF), outer, 0)

        @plsc.parallel_loop(0, B // 16, 1, unroll=2)
        def lred(i):
            d = pl.ds(i * 16, 16)
            bv = i * 272 + it17
            s = plsc.load_gather(acc, [bv])
            for r in range(1, 8):
                s = s + plsc.load_gather(acc, [bv + r])
            c = plsc.load_gather(acc, [bv + 8])
            for r in range(9, 16):
                c = c + plsc.load_gather(acc, [bv + r])
            part[0, d] = s
            part[1, d] = c

        pltpu.sync_copy(part, shared.at[sid])
        plsc.subcore_barrier()

        off = pl.multiple_of(sid * 256, 8)
        for tt in range(16):
            pltpu.sync_copy(shared.at[tt, :, pl.ds(off, 256)],
                            red_all.at[tt])

        for rr in range(4):
            kind = rr // 2
            loff = (rr % 2) * 128

            @plsc.parallel_loop(0, 8, 1, unroll=2)
            def red(i):
                d2 = pl.ds(loff + i * 16, 16)
                s = red_all[0, kind, d2]
                for tt in range(1, 16):
                    s = s + red_all[tt, kind, d2]
                obuf[rr, pl.ds(i * 16, 16)] = s

        for kind in range(2):
            row0 = pl.multiple_of(cid * 64 + kind * 32 + sid * 2, 2)
            pltpu.sync_copy(obuf.at[pl.ds(kind * 2, 2), :],
                            out_hbm.at[pl.ds(row0, 2), :])

    return body(u, batch_idx)


def _finish(partials4):
    def fin(x_ref, o_ref):
        x = x_ref[...]
        s = x[0:32] + x[64:96]
        c = x[32:64] + x[96:128]
        loss = s / jnp.maximum(c, 1.0)
        o_ref[...] = (jnp.sum(loss) * (1.0 / B)).reshape(1, 1)

    return pl.pallas_call(
        fin,
        out_shape=jax.ShapeDtypeStruct((1, 1), jnp.float32),
    )(partials4)


def kernel(pred, tgt, t, gen_flag, batch_idx, gamma):
    del t, gamma
    u = _mse_tc(pred.T, tgt.T, gen_flag.view(jnp.int8))
    partials = _sc_segment_partials(u, batch_idx)
    return _finish(partials)[0, 0]
